# baseline (speedup 1.0000x reference)
import functools

import jax
import jax.numpy as jnp
from jax import lax
from jax.experimental import pallas as pl
from jax.experimental.pallas import tpu as pltpu
from jax.experimental.pallas import tpu_sc as plsc

B = 16384
S = 4
M = 4
EMBED = 32
FEAT = 16
POKE = 48
HID = 256
NUMERIC = 24
NUM_ACTIONS = 100

NC, NS = 2, 16
NW = NC * NS
BPW = B // NW
CB = 64
NCH = BPW // CB
NBUF = 2

R = 1024

_SC_MESH = dict(core_axis_name="c", subcore_axis_name="s")


def _sc_gather_a(sp1d, ab1d, it1d, te1d, sp_table, ab_table, it_table, te_table):
    @functools.partial(
        pl.kernel,
        mesh=plsc.VectorSubcoreMesh(**_SC_MESH),
        compiler_params=pltpu.CompilerParams(use_tc_tiling_on_sc=False),
        out_type=(
            jax.ShapeDtypeStruct((B, 128), jnp.float32),
            jax.ShapeDtypeStruct((B, 128), jnp.float32),
            jax.ShapeDtypeStruct((B, 128), jnp.float32),
        ),
        scratch_types=[
            pltpu.VMEM((BPW * S,), jnp.int32),
            pltpu.VMEM((BPW * S,), jnp.int32),
            pltpu.VMEM((BPW * S,), jnp.int32),
            pltpu.VMEM((BPW * S,), jnp.int32),
            pltpu.VMEM((NBUF, CB * S, EMBED), jnp.float32),
            pltpu.VMEM((NBUF, CB * S, FEAT), jnp.float32),
            pltpu.VMEM((NBUF, CB * S, FEAT), jnp.float32),
            pltpu.VMEM((NBUF, CB * S, FEAT), jnp.float32),
            pltpu.VMEM((NBUF, CB, 128), jnp.float32),
            pltpu.VMEM((NBUF, CB, 128), jnp.float32),
            pltpu.VMEM((NBUF, CB, 128), jnp.float32),
            pltpu.SemaphoreType.DMA((NBUF,)),
            pltpu.SemaphoreType.DMA((NBUF,)),
            pltpu.SemaphoreType.DMA,
            pltpu.VMEM_SHARED((1500, EMBED), jnp.float32),
            pltpu.VMEM_SHARED((400, FEAT), jnp.float32),
            pltpu.VMEM_SHARED((600, FEAT), jnp.float32),
            pltpu.VMEM_SHARED((20, FEAT), jnp.float32),
        ],
    )
    def k(sp_ids_h, ab_ids_h, it_ids_h, te_ids_h,
          sp_th, ab_th, it_th, te_th,
          sp_o, abit_o, tep_o,
          i_sp, i_ab, i_it, i_te,
          d_sp, d_ab, d_it, d_te, o_sp, o_abit, o_tep,
          gsem, wsem, isem,
          sp_t, ab_t, it_t, te_t):
        sid = lax.axis_index("s")
        wid = sid * NC + lax.axis_index("c")
        w0 = wid * BPW

        @pl.when(sid == 0)
        def _():
            tcps = [
                pltpu.async_copy(sp_th, sp_t, isem),
                pltpu.async_copy(ab_th, ab_t, isem),
                pltpu.async_copy(it_th, it_t, isem),
                pltpu.async_copy(te_th, te_t, isem),
            ]
            for cp in tcps:
                cp.wait()

        idcps = [
            pltpu.async_copy(sp_ids_h.at[pl.ds(w0 * S, BPW * S)], i_sp, isem),
            pltpu.async_copy(ab_ids_h.at[pl.ds(w0 * S, BPW * S)], i_ab, isem),
            pltpu.async_copy(it_ids_h.at[pl.ds(w0 * S, BPW * S)], i_it, isem),
            pltpu.async_copy(te_ids_h.at[pl.ds(w0 * S, BPW * S)], i_te, isem),
        ]
        for cp in idcps:
            cp.wait()
        plsc.subcore_barrier()

        def fire_gather(c):
            b = c % NBUF
            o = c * CB * S
            return [
                pltpu.async_copy(sp_t.at[i_sp.at[pl.ds(o, CB * S)]],
                                 d_sp.at[b], gsem.at[b]),
                pltpu.async_copy(ab_t.at[i_ab.at[pl.ds(o, CB * S)]],
                                 d_ab.at[b], gsem.at[b]),
                pltpu.async_copy(it_t.at[i_it.at[pl.ds(o, CB * S)]],
                                 d_it.at[b], gsem.at[b]),
                pltpu.async_copy(te_t.at[i_te.at[pl.ds(o, CB * S)]],
                                 d_te.at[b], gsem.at[b]),
            ]

        def repack(c):
            b = c % NBUF
            dsp, dab = d_sp.at[b], d_ab.at[b]
            dit, dte = d_it.at[b], d_te.at[b]
            osp, oab, ote = o_sp.at[b], o_abit.at[b], o_tep.at[b]
            zero = jnp.zeros((16,), jnp.float32)

            def body(r, carry):
                for s in range(S):
                    p = r * S + s
                    osp[r, pl.ds(32 * s, 16)] = dsp[p, pl.ds(0, 16)]
                    osp[r, pl.ds(32 * s + 16, 16)] = dsp[p, pl.ds(16, 16)]
                    oab[r, pl.ds(16 * s, 16)] = dab[p]
                    oab[r, pl.ds(64 + 16 * s, 16)] = dit[p]
                    ote[r, pl.ds(16 * s, 16)] = dte[p]
                    ote[r, pl.ds(64 + 16 * s, 16)] = zero
                return carry

            lax.fori_loop(0, CB, body, 0)

        def fire_write(c):
            b = c % NBUF
            r0 = w0 + c * CB
            return [
                pltpu.async_copy(o_sp.at[b], sp_o.at[pl.ds(r0, CB)], wsem.at[b]),
                pltpu.async_copy(o_abit.at[b], abit_o.at[pl.ds(r0, CB)], wsem.at[b]),
                pltpu.async_copy(o_tep.at[b], tep_o.at[pl.ds(r0, CB)], wsem.at[b]),
            ]

        gcps = {c: fire_gather(c) for c in range(min(NBUF, NCH))}
        wcps = {}
        for c in range(NCH):
            for cp in gcps.pop(c):
                cp.wait()
            repack(c)
            wcps[c] = fire_write(c)
            nxt = c + NBUF
            if nxt < NCH:
                for cp in wcps.pop(c):
                    cp.wait()
                gcps[nxt] = fire_gather(nxt)
        for c, cps in wcps.items():
            for cp in cps:
                cp.wait()

    return k(sp1d, ab1d, it1d, te1d, sp_table, ab_table, it_table, te_table)


def _sc_gather_b(mv1d, mv_table):
    @functools.partial(
        pl.kernel,
        mesh=plsc.VectorSubcoreMesh(**_SC_MESH),
        compiler_params=pltpu.CompilerParams(use_tc_tiling_on_sc=False),
        out_type=jax.ShapeDtypeStruct((B, 128), jnp.float32),
        scratch_types=[
            pltpu.VMEM((BPW * S * M,), jnp.int32),
            pltpu.VMEM((NBUF, CB * S * M, FEAT), jnp.float32),
            pltpu.VMEM((NBUF, CB, 128), jnp.float32),
            pltpu.SemaphoreType.DMA((NBUF,)),
            pltpu.SemaphoreType.DMA((NBUF,)),
            pltpu.SemaphoreType.DMA,
            pltpu.VMEM_SHARED((1000, FEAT), jnp.float32),
        ],
    )
    def k(mv_ids_h, mv_th, mvp_o, i_mv, d_mv, o_mv, gsem, wsem, isem, mv_t):
        sid = lax.axis_index("s")
        wid = sid * NC + lax.axis_index("c")
        w0 = wid * BPW

        @pl.when(sid == 0)
        def _():
            pltpu.async_copy(mv_th, mv_t, isem).wait()

        pltpu.async_copy(
            mv_ids_h.at[pl.ds(w0 * S * M, BPW * S * M)], i_mv, isem).wait()
        plsc.subcore_barrier()

        def fire_gather(c):
            b = c % NBUF
            om = c * CB * S * M
            return [pltpu.async_copy(mv_t.at[i_mv.at[pl.ds(om, CB * S * M)]],
                                     d_mv.at[b], gsem.at[b])]

        def repack(c):
            b = c % NBUF
            dmv, omv = d_mv.at[b], o_mv.at[b]
            zero = jnp.zeros((16,), jnp.float32)

            def body(r, carry):
                for s in range(S):
                    p = (r * S + s) * M
                    acc = dmv[p] + dmv[p + 1] + dmv[p + 2] + dmv[p + 3]
                    omv[r, pl.ds(16 * s, 16)] = acc
                    omv[r, pl.ds(64 + 16 * s, 16)] = zero
                return carry

            lax.fori_loop(0, CB, body, 0)

        def fire_write(c):
            b = c % NBUF
            return [pltpu.async_copy(o_mv.at[b],
                                     mvp_o.at[pl.ds(w0 + c * CB, CB)],
                                     wsem.at[b])]

        gcps = {c: fire_gather(c) for c in range(min(NBUF, NCH))}
        wcps = {}
        for c in range(NCH):
            for cp in gcps.pop(c):
                cp.wait()
            repack(c)
            wcps[c] = fire_write(c)
            nxt = c + NBUF
            if nxt < NCH:
                for cp in wcps.pop(c):
                    cp.wait()
                gcps[nxt] = fire_gather(nxt)
        for c, cps in wcps.items():
            for cp in cps:
                cp.wait()

    return k(mv1d, mv_table)


def _mlp_body(sp_r, abit_r, tep_r, mvp_r, num_r,
              wsp_r, wabit_r, wtep_r, wmvp_r, bpe_r,
              w1f_r, w1n_r, b1_r, w2_r, b2_r, w3_r, b3_r,
              wv1_r, bv1_r, wv2_r, bv2_r,
              wp1x_r, wp1s_r, bp1_r, wp2_r, bp2_r,
              v_o, p0_o, p1_o):
    dot = functools.partial(jnp.dot, preferred_element_type=jnp.float32)
    bf = lambda x: x.astype(jnp.bfloat16)
    enc = dot(bf(sp_r[:]), wsp_r[:]) + dot(bf(abit_r[:]), wabit_r[:]) \
        + dot(bf(tep_r[:]), wtep_r[:]) + dot(bf(mvp_r[:]), wmvp_r[:]) + bpe_r[:]
    encb = bf(jnp.maximum(enc, 0.0))
    x = jnp.maximum(dot(encb, w1f_r[:]) + dot(bf(num_r[:]), w1n_r[:]) + b1_r[:],
                    0.0)
    x = bf(x)
    x = bf(jnp.maximum(dot(x, w2_r[:]) + b2_r[:], 0.0))
    x = bf(jnp.maximum(dot(x, w3_r[:]) + b3_r[:], 0.0))
    v = dot(bf(jnp.maximum(dot(x, wv1_r[:]) + bv1_r[:], 0.0)), wv2_r[:]) \
        + bv2_r[:]
    v_o[:] = v[:, 0]
    for i, p_o in ((0, p0_o), (1, p1_o)):
        slot = encb[:, i * POKE:(i + 1) * POKE]
        h = jnp.maximum(dot(x, wp1x_r[:]) + dot(slot, wp1s_r[:]) + bp1_r[:], 0.0)
        p_o[:] = dot(bf(h), wp2_r[:]) + bp2_r[:]


def kernel(species_ids, move_ids, ability_ids, item_ids, tera_ids, numeric, sp_table, mv_table, ab_table, it_table, te_table, W_pe, b_pe, W1, b1, g1, be1, W2, b2, g2, be2, W3, b3, g3, be3, Wv1, bv1, Wv2, bv2, Wp1, bp1, Wp2, bp2):
    f32 = jnp.float32
    mvp = _sc_gather_b(move_ids.reshape(-1), mv_table)
    sp1d, mvp = lax.optimization_barrier((species_ids.reshape(-1), mvp))
    sp2, abit, tep = _sc_gather_a(
        sp1d, ability_ids.reshape(-1),
        item_ids.reshape(-1), tera_ids.reshape(-1),
        sp_table, ab_table, it_table, te_table)

    def bd16(blk_lo, blk_hi):
        z = jnp.zeros((128, S * POKE), f32)
        for s in range(S):
            z = z.at[16 * s:16 * (s + 1), POKE * s:POKE * (s + 1)].set(blk_lo)
            if blk_hi is not None:
                z = z.at[64 + 16 * s:64 + 16 * (s + 1),
                         POKE * s:POKE * (s + 1)].set(blk_hi)
        return z

    wsp = jnp.zeros((128, S * POKE), f32)
    for s in range(S):
        wsp = wsp.at[32 * s:32 * (s + 1), POKE * s:POKE * (s + 1)].set(W_pe[0:32])
    wabit = bd16(W_pe[48:64], W_pe[64:80])
    wtep = bd16(W_pe[80:96], None)
    wmvp = bd16(W_pe[32:48], None)
    bpe = jnp.tile(b_pe, S).reshape(1, S * POKE)

    inv = 1.0 / jnp.sqrt(1.0 + 1e-5)
    w1 = W1 * (g1 * inv)[None, :]
    b1f = (b1 * g1 * inv + be1).reshape(1, HID)
    w2 = W2 * (g2 * inv)[None, :]
    b2f = (b2 * g2 * inv + be2).reshape(1, HID)
    w3 = W3 * (g3 * inv)[None, :]
    b3f = (b3 * g3 * inv + be3).reshape(1, HID // 2)
    w1f, w1n = w1[:S * POKE], w1[S * POKE:]

    grid = (B // R,)
    bf16 = jnp.bfloat16
    row = lambda c: pl.BlockSpec((R, c), lambda i: (i, 0))
    full = lambda a: pl.BlockSpec(a.shape, lambda i: (0,) * a.ndim)
    wargs = (wsp.astype(bf16), wabit.astype(bf16), wtep.astype(bf16),
             wmvp.astype(bf16), bpe,
             w1f.astype(bf16), w1n.astype(bf16), b1f,
             w2.astype(bf16), b2f, w3.astype(bf16), b3f,
             Wv1.astype(bf16), bv1.reshape(1, 64), Wv2.astype(bf16),
             bv2.reshape(1, 1),
             Wp1[:HID // 2].astype(bf16), Wp1[HID // 2:].astype(bf16),
             bp1.reshape(1, HID // 2),
             Wp2.astype(bf16), bp2.reshape(1, NUM_ACTIONS))
    v, p0, p1 = pl.pallas_call(
        _mlp_body,
        grid=grid,
        in_specs=[row(128), row(128), row(128), row(128), row(NUMERIC)]
                 + [full(a) for a in wargs],
        out_specs=[pl.BlockSpec((R,), lambda i: (i,)),
                   row(NUM_ACTIONS), row(NUM_ACTIONS)],
        out_shape=[jax.ShapeDtypeStruct((B,), f32),
                   jax.ShapeDtypeStruct((B, NUM_ACTIONS), f32),
                   jax.ShapeDtypeStruct((B, NUM_ACTIONS), f32)],
        compiler_params=pltpu.CompilerParams(
            dimension_semantics=("parallel",)),
    )(sp2, abit, tep, mvp, numeric, *wargs)
    return (v, p0, p1)


# speedup vs baseline: 1.0185x; 1.0185x over previous
import functools

import jax
import jax.numpy as jnp
from jax import lax
from jax.experimental import pallas as pl
from jax.experimental.pallas import tpu as pltpu
from jax.experimental.pallas import tpu_sc as plsc

B = 16384
S = 4
M = 4
EMBED = 32
FEAT = 16
POKE = 48
HID = 256
NUMERIC = 24
NUM_ACTIONS = 100

NC, NS = 2, 16
NW = NC * NS
BPW = B // NW
CB = 64
NCH = BPW // CB
NBUF = 2

R = 2048

_SC_MESH = dict(core_axis_name="c", subcore_axis_name="s")


def _sc_gather_a(sp1d, ab1d, it1d, te1d, sp_table, ab_table, it_table, te_table):
    @functools.partial(
        pl.kernel,
        mesh=plsc.VectorSubcoreMesh(**_SC_MESH),
        compiler_params=pltpu.CompilerParams(use_tc_tiling_on_sc=False),
        out_type=(
            jax.ShapeDtypeStruct((B, 128), jnp.float32),
            jax.ShapeDtypeStruct((B, 128), jnp.float32),
            jax.ShapeDtypeStruct((B, 128), jnp.float32),
        ),
        scratch_types=[
            pltpu.VMEM((BPW * S,), jnp.int32),
            pltpu.VMEM((BPW * S,), jnp.int32),
            pltpu.VMEM((BPW * S,), jnp.int32),
            pltpu.VMEM((BPW * S,), jnp.int32),
            pltpu.VMEM((NBUF, CB * S, EMBED), jnp.float32),
            pltpu.VMEM((NBUF, CB * S, FEAT), jnp.float32),
            pltpu.VMEM((NBUF, CB * S, FEAT), jnp.float32),
            pltpu.VMEM((NBUF, CB * S, FEAT), jnp.float32),
            pltpu.VMEM((NBUF, CB, 128), jnp.float32),
            pltpu.VMEM((NBUF, CB, 128), jnp.float32),
            pltpu.VMEM((NBUF, CB, 128), jnp.float32),
            pltpu.SemaphoreType.DMA((NBUF,)),
            pltpu.SemaphoreType.DMA((NBUF,)),
            pltpu.SemaphoreType.DMA,
            pltpu.VMEM_SHARED((1500, EMBED), jnp.float32),
            pltpu.VMEM_SHARED((400, FEAT), jnp.float32),
            pltpu.VMEM_SHARED((600, FEAT), jnp.float32),
            pltpu.VMEM_SHARED((20, FEAT), jnp.float32),
        ],
    )
    def k(sp_ids_h, ab_ids_h, it_ids_h, te_ids_h,
          sp_th, ab_th, it_th, te_th,
          sp_o, abit_o, tep_o,
          i_sp, i_ab, i_it, i_te,
          d_sp, d_ab, d_it, d_te, o_sp, o_abit, o_tep,
          gsem, wsem, isem,
          sp_t, ab_t, it_t, te_t):
        sid = lax.axis_index("s")
        wid = sid * NC + lax.axis_index("c")
        w0 = wid * BPW

        @pl.when(sid == 0)
        def _():
            tcps = [
                pltpu.async_copy(sp_th, sp_t, isem),
                pltpu.async_copy(ab_th, ab_t, isem),
                pltpu.async_copy(it_th, it_t, isem),
                pltpu.async_copy(te_th, te_t, isem),
            ]
            for cp in tcps:
                cp.wait()

        idcps = [
            pltpu.async_copy(sp_ids_h.at[pl.ds(w0 * S, BPW * S)], i_sp, isem),
            pltpu.async_copy(ab_ids_h.at[pl.ds(w0 * S, BPW * S)], i_ab, isem),
            pltpu.async_copy(it_ids_h.at[pl.ds(w0 * S, BPW * S)], i_it, isem),
            pltpu.async_copy(te_ids_h.at[pl.ds(w0 * S, BPW * S)], i_te, isem),
        ]
        for cp in idcps:
            cp.wait()
        plsc.subcore_barrier()

        def fire_gather(c):
            b = c % NBUF
            o = c * CB * S
            return [
                pltpu.async_copy(sp_t.at[i_sp.at[pl.ds(o, CB * S)]],
                                 d_sp.at[b], gsem.at[b]),
                pltpu.async_copy(ab_t.at[i_ab.at[pl.ds(o, CB * S)]],
                                 d_ab.at[b], gsem.at[b]),
                pltpu.async_copy(it_t.at[i_it.at[pl.ds(o, CB * S)]],
                                 d_it.at[b], gsem.at[b]),
                pltpu.async_copy(te_t.at[i_te.at[pl.ds(o, CB * S)]],
                                 d_te.at[b], gsem.at[b]),
            ]

        def repack(c):
            b = c % NBUF
            dsp, dab = d_sp.at[b], d_ab.at[b]
            dit, dte = d_it.at[b], d_te.at[b]
            osp, oab, ote = o_sp.at[b], o_abit.at[b], o_tep.at[b]
            zero = jnp.zeros((16,), jnp.float32)

            def body(r, carry):
                for s in range(S):
                    p = r * S + s
                    osp[r, pl.ds(32 * s, 16)] = dsp[p, pl.ds(0, 16)]
                    osp[r, pl.ds(32 * s + 16, 16)] = dsp[p, pl.ds(16, 16)]
                    oab[r, pl.ds(16 * s, 16)] = dab[p]
                    oab[r, pl.ds(64 + 16 * s, 16)] = dit[p]
                    ote[r, pl.ds(16 * s, 16)] = dte[p]
                    ote[r, pl.ds(64 + 16 * s, 16)] = zero
                return carry

            lax.fori_loop(0, CB, body, 0)

        def fire_write(c):
            b = c % NBUF
            r0 = w0 + c * CB
            return [
                pltpu.async_copy(o_sp.at[b], sp_o.at[pl.ds(r0, CB)], wsem.at[b]),
                pltpu.async_copy(o_abit.at[b], abit_o.at[pl.ds(r0, CB)], wsem.at[b]),
                pltpu.async_copy(o_tep.at[b], tep_o.at[pl.ds(r0, CB)], wsem.at[b]),
            ]

        gcps = {c: fire_gather(c) for c in range(min(NBUF, NCH))}
        wcps = {}
        for c in range(NCH):
            for cp in gcps.pop(c):
                cp.wait()
            repack(c)
            wcps[c] = fire_write(c)
            nxt = c + NBUF
            if nxt < NCH:
                for cp in wcps.pop(c):
                    cp.wait()
                gcps[nxt] = fire_gather(nxt)
        for c, cps in wcps.items():
            for cp in cps:
                cp.wait()

    return k(sp1d, ab1d, it1d, te1d, sp_table, ab_table, it_table, te_table)


def _sc_gather_b(mv1d, mv_table):
    @functools.partial(
        pl.kernel,
        mesh=plsc.VectorSubcoreMesh(**_SC_MESH),
        compiler_params=pltpu.CompilerParams(use_tc_tiling_on_sc=False),
        out_type=jax.ShapeDtypeStruct((B, 128), jnp.float32),
        scratch_types=[
            pltpu.VMEM((BPW * S * M,), jnp.int32),
            pltpu.VMEM((NBUF, CB * S * M, FEAT), jnp.float32),
            pltpu.VMEM((NBUF, CB, 128), jnp.float32),
            pltpu.SemaphoreType.DMA((NBUF,)),
            pltpu.SemaphoreType.DMA((NBUF,)),
            pltpu.SemaphoreType.DMA,
            pltpu.VMEM_SHARED((1000, FEAT), jnp.float32),
        ],
    )
    def k(mv_ids_h, mv_th, mvp_o, i_mv, d_mv, o_mv, gsem, wsem, isem, mv_t):
        sid = lax.axis_index("s")
        wid = sid * NC + lax.axis_index("c")
        w0 = wid * BPW

        @pl.when(sid == 0)
        def _():
            pltpu.async_copy(mv_th, mv_t, isem).wait()

        pltpu.async_copy(
            mv_ids_h.at[pl.ds(w0 * S * M, BPW * S * M)], i_mv, isem).wait()
        plsc.subcore_barrier()

        def fire_gather(c):
            b = c % NBUF
            om = c * CB * S * M
            return [pltpu.async_copy(mv_t.at[i_mv.at[pl.ds(om, CB * S * M)]],
                                     d_mv.at[b], gsem.at[b])]

        def repack(c):
            b = c % NBUF
            dmv, omv = d_mv.at[b], o_mv.at[b]
            zero = jnp.zeros((16,), jnp.float32)

            def body(r, carry):
                for s in range(S):
                    p = (r * S + s) * M
                    acc = dmv[p] + dmv[p + 1] + dmv[p + 2] + dmv[p + 3]
                    omv[r, pl.ds(16 * s, 16)] = acc
                    omv[r, pl.ds(64 + 16 * s, 16)] = zero
                return carry

            lax.fori_loop(0, CB, body, 0)

        def fire_write(c):
            b = c % NBUF
            return [pltpu.async_copy(o_mv.at[b],
                                     mvp_o.at[pl.ds(w0 + c * CB, CB)],
                                     wsem.at[b])]

        gcps = {c: fire_gather(c) for c in range(min(NBUF, NCH))}
        wcps = {}
        for c in range(NCH):
            for cp in gcps.pop(c):
                cp.wait()
            repack(c)
            wcps[c] = fire_write(c)
            nxt = c + NBUF
            if nxt < NCH:
                for cp in wcps.pop(c):
                    cp.wait()
                gcps[nxt] = fire_gather(nxt)
        for c, cps in wcps.items():
            for cp in cps:
                cp.wait()

    return k(mv1d, mv_table)


def _mlp_body(sp_r, abit_r, tep_r, mvp_r, num_r,
              wsp_r, wabit_r, wtep_r, wmvp_r, bpe_r,
              w1f_r, w1n_r, b1_r, w2_r, b2_r, w3_r, b3_r,
              wv1_r, bv1_r, wv2_r, bv2_r,
              wp1x_r, wp1s_r, bp1_r, wp2_r, bp2_r,
              v_o, p0_o, p1_o):
    dot = functools.partial(jnp.dot, preferred_element_type=jnp.float32)
    bf = lambda x: x.astype(jnp.bfloat16)
    enc = dot(bf(sp_r[:]), wsp_r[:]) + dot(bf(abit_r[:]), wabit_r[:]) \
        + dot(bf(tep_r[:]), wtep_r[:]) + dot(bf(mvp_r[:]), wmvp_r[:]) + bpe_r[:]
    encb = bf(jnp.maximum(enc, 0.0))
    x = jnp.maximum(dot(encb, w1f_r[:]) + dot(bf(num_r[:]), w1n_r[:]) + b1_r[:],
                    0.0)
    x = bf(x)
    x = bf(jnp.maximum(dot(x, w2_r[:]) + b2_r[:], 0.0))
    x = bf(jnp.maximum(dot(x, w3_r[:]) + b3_r[:], 0.0))
    v = dot(bf(jnp.maximum(dot(x, wv1_r[:]) + bv1_r[:], 0.0)), wv2_r[:]) \
        + bv2_r[:]
    v_o[:] = v[:, 0]
    for i, p_o in ((0, p0_o), (1, p1_o)):
        slot = encb[:, i * POKE:(i + 1) * POKE]
        h = jnp.maximum(dot(x, wp1x_r[:]) + dot(slot, wp1s_r[:]) + bp1_r[:], 0.0)
        p_o[:] = dot(bf(h), wp2_r[:]) + bp2_r[:]


def kernel(species_ids, move_ids, ability_ids, item_ids, tera_ids, numeric, sp_table, mv_table, ab_table, it_table, te_table, W_pe, b_pe, W1, b1, g1, be1, W2, b2, g2, be2, W3, b3, g3, be3, Wv1, bv1, Wv2, bv2, Wp1, bp1, Wp2, bp2):
    f32 = jnp.float32
    mvp = _sc_gather_b(move_ids.reshape(-1), mv_table)
    sp1d, mvp = lax.optimization_barrier((species_ids.reshape(-1), mvp))
    sp2, abit, tep = _sc_gather_a(
        sp1d, ability_ids.reshape(-1),
        item_ids.reshape(-1), tera_ids.reshape(-1),
        sp_table, ab_table, it_table, te_table)

    def bd16(blk_lo, blk_hi):
        z = jnp.zeros((128, S * POKE), f32)
        for s in range(S):
            z = z.at[16 * s:16 * (s + 1), POKE * s:POKE * (s + 1)].set(blk_lo)
            if blk_hi is not None:
                z = z.at[64 + 16 * s:64 + 16 * (s + 1),
                         POKE * s:POKE * (s + 1)].set(blk_hi)
        return z

    wsp = jnp.zeros((128, S * POKE), f32)
    for s in range(S):
        wsp = wsp.at[32 * s:32 * (s + 1), POKE * s:POKE * (s + 1)].set(W_pe[0:32])
    wabit = bd16(W_pe[48:64], W_pe[64:80])
    wtep = bd16(W_pe[80:96], None)
    wmvp = bd16(W_pe[32:48], None)
    bpe = jnp.tile(b_pe, S).reshape(1, S * POKE)

    inv = 1.0 / jnp.sqrt(1.0 + 1e-5)
    w1 = W1 * (g1 * inv)[None, :]
    b1f = (b1 * g1 * inv + be1).reshape(1, HID)
    w2 = W2 * (g2 * inv)[None, :]
    b2f = (b2 * g2 * inv + be2).reshape(1, HID)
    w3 = W3 * (g3 * inv)[None, :]
    b3f = (b3 * g3 * inv + be3).reshape(1, HID // 2)
    w1f, w1n = w1[:S * POKE], w1[S * POKE:]

    grid = (B // R,)
    bf16 = jnp.bfloat16
    row = lambda c: pl.BlockSpec((R, c), lambda i: (i, 0))
    full = lambda a: pl.BlockSpec(a.shape, lambda i: (0,) * a.ndim)
    wargs = (wsp.astype(bf16), wabit.astype(bf16), wtep.astype(bf16),
             wmvp.astype(bf16), bpe,
             w1f.astype(bf16), w1n.astype(bf16), b1f,
             w2.astype(bf16), b2f, w3.astype(bf16), b3f,
             Wv1.astype(bf16), bv1.reshape(1, 64), Wv2.astype(bf16),
             bv2.reshape(1, 1),
             Wp1[:HID // 2].astype(bf16), Wp1[HID // 2:].astype(bf16),
             bp1.reshape(1, HID // 2),
             Wp2.astype(bf16), bp2.reshape(1, NUM_ACTIONS))
    v, p0, p1 = pl.pallas_call(
        _mlp_body,
        grid=grid,
        in_specs=[row(128), row(128), row(128), row(128), row(NUMERIC)]
                 + [full(a) for a in wargs],
        out_specs=[pl.BlockSpec((R,), lambda i: (i,)),
                   row(NUM_ACTIONS), row(NUM_ACTIONS)],
        out_shape=[jax.ShapeDtypeStruct((B,), f32),
                   jax.ShapeDtypeStruct((B, NUM_ACTIONS), f32),
                   jax.ShapeDtypeStruct((B, NUM_ACTIONS), f32)],
        compiler_params=pltpu.CompilerParams(
            dimension_semantics=("parallel",)),
    )(sp2, abit, tep, mvp, numeric, *wargs)
    return (v, p0, p1)


# speedup vs baseline: 1.0195x; 1.0010x over previous
import functools

import jax
import jax.numpy as jnp
from jax import lax
from jax.experimental import pallas as pl
from jax.experimental.pallas import tpu as pltpu
from jax.experimental.pallas import tpu_sc as plsc

B = 16384
S = 4
M = 4
EMBED = 32
FEAT = 16
POKE = 48
HID = 256
NUMERIC = 24
NUM_ACTIONS = 100

NC, NS = 2, 16
NW = NC * NS
BPW = B // NW
CB = 64
NCH = BPW // CB
NBUF = 2

R = 4096

_SC_MESH = dict(core_axis_name="c", subcore_axis_name="s")


def _sc_gather_a(sp1d, ab1d, it1d, te1d, sp_table, ab_table, it_table, te_table):
    @functools.partial(
        pl.kernel,
        mesh=plsc.VectorSubcoreMesh(**_SC_MESH),
        compiler_params=pltpu.CompilerParams(use_tc_tiling_on_sc=False),
        out_type=(
            jax.ShapeDtypeStruct((B, 128), jnp.float32),
            jax.ShapeDtypeStruct((B, 128), jnp.float32),
            jax.ShapeDtypeStruct((B, 128), jnp.float32),
        ),
        scratch_types=[
            pltpu.VMEM((BPW * S,), jnp.int32),
            pltpu.VMEM((BPW * S,), jnp.int32),
            pltpu.VMEM((BPW * S,), jnp.int32),
            pltpu.VMEM((BPW * S,), jnp.int32),
            pltpu.VMEM((NBUF, CB * S, EMBED), jnp.float32),
            pltpu.VMEM((NBUF, CB * S, FEAT), jnp.float32),
            pltpu.VMEM((NBUF, CB * S, FEAT), jnp.float32),
            pltpu.VMEM((NBUF, CB * S, FEAT), jnp.float32),
            pltpu.VMEM((NBUF, CB, 128), jnp.float32),
            pltpu.VMEM((NBUF, CB, 128), jnp.float32),
            pltpu.VMEM((NBUF, CB, 128), jnp.float32),
            pltpu.SemaphoreType.DMA((NBUF,)),
            pltpu.SemaphoreType.DMA((NBUF,)),
            pltpu.SemaphoreType.DMA,
            pltpu.VMEM_SHARED((1500, EMBED), jnp.float32),
            pltpu.VMEM_SHARED((400, FEAT), jnp.float32),
            pltpu.VMEM_SHARED((600, FEAT), jnp.float32),
            pltpu.VMEM_SHARED((20, FEAT), jnp.float32),
        ],
    )
    def k(sp_ids_h, ab_ids_h, it_ids_h, te_ids_h,
          sp_th, ab_th, it_th, te_th,
          sp_o, abit_o, tep_o,
          i_sp, i_ab, i_it, i_te,
          d_sp, d_ab, d_it, d_te, o_sp, o_abit, o_tep,
          gsem, wsem, isem,
          sp_t, ab_t, it_t, te_t):
        sid = lax.axis_index("s")
        wid = sid * NC + lax.axis_index("c")
        w0 = wid * BPW

        @pl.when(sid == 0)
        def _():
            tcps = [
                pltpu.async_copy(sp_th, sp_t, isem),
                pltpu.async_copy(ab_th, ab_t, isem),
                pltpu.async_copy(it_th, it_t, isem),
                pltpu.async_copy(te_th, te_t, isem),
            ]
            for cp in tcps:
                cp.wait()

        idcps = [
            pltpu.async_copy(sp_ids_h.at[pl.ds(w0 * S, BPW * S)], i_sp, isem),
            pltpu.async_copy(ab_ids_h.at[pl.ds(w0 * S, BPW * S)], i_ab, isem),
            pltpu.async_copy(it_ids_h.at[pl.ds(w0 * S, BPW * S)], i_it, isem),
            pltpu.async_copy(te_ids_h.at[pl.ds(w0 * S, BPW * S)], i_te, isem),
        ]
        for cp in idcps:
            cp.wait()
        plsc.subcore_barrier()

        def fire_gather(c):
            b = c % NBUF
            o = c * CB * S
            return [
                pltpu.async_copy(sp_t.at[i_sp.at[pl.ds(o, CB * S)]],
                                 d_sp.at[b], gsem.at[b]),
                pltpu.async_copy(ab_t.at[i_ab.at[pl.ds(o, CB * S)]],
                                 d_ab.at[b], gsem.at[b]),
                pltpu.async_copy(it_t.at[i_it.at[pl.ds(o, CB * S)]],
                                 d_it.at[b], gsem.at[b]),
                pltpu.async_copy(te_t.at[i_te.at[pl.ds(o, CB * S)]],
                                 d_te.at[b], gsem.at[b]),
            ]

        def repack(c):
            b = c % NBUF
            dsp, dab = d_sp.at[b], d_ab.at[b]
            dit, dte = d_it.at[b], d_te.at[b]
            osp, oab, ote = o_sp.at[b], o_abit.at[b], o_tep.at[b]
            zero = jnp.zeros((16,), jnp.float32)

            def body(r, carry):
                for s in range(S):
                    p = r * S + s
                    osp[r, pl.ds(32 * s, 16)] = dsp[p, pl.ds(0, 16)]
                    osp[r, pl.ds(32 * s + 16, 16)] = dsp[p, pl.ds(16, 16)]
                    oab[r, pl.ds(16 * s, 16)] = dab[p]
                    oab[r, pl.ds(64 + 16 * s, 16)] = dit[p]
                    ote[r, pl.ds(16 * s, 16)] = dte[p]
                    ote[r, pl.ds(64 + 16 * s, 16)] = zero
                return carry

            lax.fori_loop(0, CB, body, 0)

        def fire_write(c):
            b = c % NBUF
            r0 = w0 + c * CB
            return [
                pltpu.async_copy(o_sp.at[b], sp_o.at[pl.ds(r0, CB)], wsem.at[b]),
                pltpu.async_copy(o_abit.at[b], abit_o.at[pl.ds(r0, CB)], wsem.at[b]),
                pltpu.async_copy(o_tep.at[b], tep_o.at[pl.ds(r0, CB)], wsem.at[b]),
            ]

        gcps = {c: fire_gather(c) for c in range(min(NBUF, NCH))}
        wcps = {}
        for c in range(NCH):
            for cp in gcps.pop(c):
                cp.wait()
            repack(c)
            wcps[c] = fire_write(c)
            nxt = c + NBUF
            if nxt < NCH:
                for cp in wcps.pop(c):
                    cp.wait()
                gcps[nxt] = fire_gather(nxt)
        for c, cps in wcps.items():
            for cp in cps:
                cp.wait()

    return k(sp1d, ab1d, it1d, te1d, sp_table, ab_table, it_table, te_table)


def _sc_gather_b(mv1d, mv_table):
    @functools.partial(
        pl.kernel,
        mesh=plsc.VectorSubcoreMesh(**_SC_MESH),
        compiler_params=pltpu.CompilerParams(use_tc_tiling_on_sc=False),
        out_type=jax.ShapeDtypeStruct((B, 128), jnp.float32),
        scratch_types=[
            pltpu.VMEM((BPW * S * M,), jnp.int32),
            pltpu.VMEM((NBUF, CB * S * M, FEAT), jnp.float32),
            pltpu.VMEM((NBUF, CB, 128), jnp.float32),
            pltpu.SemaphoreType.DMA((NBUF,)),
            pltpu.SemaphoreType.DMA((NBUF,)),
            pltpu.SemaphoreType.DMA,
            pltpu.VMEM_SHARED((1000, FEAT), jnp.float32),
        ],
    )
    def k(mv_ids_h, mv_th, mvp_o, i_mv, d_mv, o_mv, gsem, wsem, isem, mv_t):
        sid = lax.axis_index("s")
        wid = sid * NC + lax.axis_index("c")
        w0 = wid * BPW

        @pl.when(sid == 0)
        def _():
            pltpu.async_copy(mv_th, mv_t, isem).wait()

        pltpu.async_copy(
            mv_ids_h.at[pl.ds(w0 * S * M, BPW * S * M)], i_mv, isem).wait()
        plsc.subcore_barrier()

        def fire_gather(c):
            b = c % NBUF
            om = c * CB * S * M
            return [pltpu.async_copy(mv_t.at[i_mv.at[pl.ds(om, CB * S * M)]],
                                     d_mv.at[b], gsem.at[b])]

        def repack(c):
            b = c % NBUF
            dmv, omv = d_mv.at[b], o_mv.at[b]
            zero = jnp.zeros((16,), jnp.float32)

            def body(r, carry):
                for s in range(S):
                    p = (r * S + s) * M
                    acc = dmv[p] + dmv[p + 1] + dmv[p + 2] + dmv[p + 3]
                    omv[r, pl.ds(16 * s, 16)] = acc
                    omv[r, pl.ds(64 + 16 * s, 16)] = zero
                return carry

            lax.fori_loop(0, CB, body, 0)

        def fire_write(c):
            b = c % NBUF
            return [pltpu.async_copy(o_mv.at[b],
                                     mvp_o.at[pl.ds(w0 + c * CB, CB)],
                                     wsem.at[b])]

        gcps = {c: fire_gather(c) for c in range(min(NBUF, NCH))}
        wcps = {}
        for c in range(NCH):
            for cp in gcps.pop(c):
                cp.wait()
            repack(c)
            wcps[c] = fire_write(c)
            nxt = c + NBUF
            if nxt < NCH:
                for cp in wcps.pop(c):
                    cp.wait()
                gcps[nxt] = fire_gather(nxt)
        for c, cps in wcps.items():
            for cp in cps:
                cp.wait()

    return k(mv1d, mv_table)


def _mlp_body(sp_r, abit_r, tep_r, mvp_r, num_r,
              wsp_r, wabit_r, wtep_r, wmvp_r, bpe_r,
              w1f_r, w1n_r, b1_r, w2_r, b2_r, w3_r, b3_r,
              wv1_r, bv1_r, wv2_r, bv2_r,
              wp1x_r, wp1s_r, bp1_r, wp2_r, bp2_r,
              v_o, p0_o, p1_o):
    dot = functools.partial(jnp.dot, preferred_element_type=jnp.float32)
    bf = lambda x: x.astype(jnp.bfloat16)
    enc = dot(bf(sp_r[:]), wsp_r[:]) + dot(bf(abit_r[:]), wabit_r[:]) \
        + dot(bf(tep_r[:]), wtep_r[:]) + dot(bf(mvp_r[:]), wmvp_r[:]) + bpe_r[:]
    encb = bf(jnp.maximum(enc, 0.0))
    x = jnp.maximum(dot(encb, w1f_r[:]) + dot(bf(num_r[:]), w1n_r[:]) + b1_r[:],
                    0.0)
    x = bf(x)
    x = bf(jnp.maximum(dot(x, w2_r[:]) + b2_r[:], 0.0))
    x = bf(jnp.maximum(dot(x, w3_r[:]) + b3_r[:], 0.0))
    v = dot(bf(jnp.maximum(dot(x, wv1_r[:]) + bv1_r[:], 0.0)), wv2_r[:]) \
        + bv2_r[:]
    v_o[:] = v[:, 0]
    for i, p_o in ((0, p0_o), (1, p1_o)):
        slot = encb[:, i * POKE:(i + 1) * POKE]
        h = jnp.maximum(dot(x, wp1x_r[:]) + dot(slot, wp1s_r[:]) + bp1_r[:], 0.0)
        p_o[:] = dot(bf(h), wp2_r[:]) + bp2_r[:]


def kernel(species_ids, move_ids, ability_ids, item_ids, tera_ids, numeric, sp_table, mv_table, ab_table, it_table, te_table, W_pe, b_pe, W1, b1, g1, be1, W2, b2, g2, be2, W3, b3, g3, be3, Wv1, bv1, Wv2, bv2, Wp1, bp1, Wp2, bp2):
    f32 = jnp.float32
    mvp = _sc_gather_b(move_ids.reshape(-1), mv_table)
    sp1d, mvp = lax.optimization_barrier((species_ids.reshape(-1), mvp))
    sp2, abit, tep = _sc_gather_a(
        sp1d, ability_ids.reshape(-1),
        item_ids.reshape(-1), tera_ids.reshape(-1),
        sp_table, ab_table, it_table, te_table)

    def bd16(blk_lo, blk_hi):
        z = jnp.zeros((128, S * POKE), f32)
        for s in range(S):
            z = z.at[16 * s:16 * (s + 1), POKE * s:POKE * (s + 1)].set(blk_lo)
            if blk_hi is not None:
                z = z.at[64 + 16 * s:64 + 16 * (s + 1),
                         POKE * s:POKE * (s + 1)].set(blk_hi)
        return z

    wsp = jnp.zeros((128, S * POKE), f32)
    for s in range(S):
        wsp = wsp.at[32 * s:32 * (s + 1), POKE * s:POKE * (s + 1)].set(W_pe[0:32])
    wabit = bd16(W_pe[48:64], W_pe[64:80])
    wtep = bd16(W_pe[80:96], None)
    wmvp = bd16(W_pe[32:48], None)
    bpe = jnp.tile(b_pe, S).reshape(1, S * POKE)

    inv = 1.0 / jnp.sqrt(1.0 + 1e-5)
    w1 = W1 * (g1 * inv)[None, :]
    b1f = (b1 * g1 * inv + be1).reshape(1, HID)
    w2 = W2 * (g2 * inv)[None, :]
    b2f = (b2 * g2 * inv + be2).reshape(1, HID)
    w3 = W3 * (g3 * inv)[None, :]
    b3f = (b3 * g3 * inv + be3).reshape(1, HID // 2)
    w1f, w1n = w1[:S * POKE], w1[S * POKE:]

    grid = (B // R,)
    bf16 = jnp.bfloat16
    row = lambda c: pl.BlockSpec((R, c), lambda i: (i, 0))
    full = lambda a: pl.BlockSpec(a.shape, lambda i: (0,) * a.ndim)
    wargs = (wsp.astype(bf16), wabit.astype(bf16), wtep.astype(bf16),
             wmvp.astype(bf16), bpe,
             w1f.astype(bf16), w1n.astype(bf16), b1f,
             w2.astype(bf16), b2f, w3.astype(bf16), b3f,
             Wv1.astype(bf16), bv1.reshape(1, 64), Wv2.astype(bf16),
             bv2.reshape(1, 1),
             Wp1[:HID // 2].astype(bf16), Wp1[HID // 2:].astype(bf16),
             bp1.reshape(1, HID // 2),
             Wp2.astype(bf16), bp2.reshape(1, NUM_ACTIONS))
    v, p0, p1 = pl.pallas_call(
        _mlp_body,
        grid=grid,
        in_specs=[row(128), row(128), row(128), row(128), row(NUMERIC)]
                 + [full(a) for a in wargs],
        out_specs=[pl.BlockSpec((R,), lambda i: (i,)),
                   row(NUM_ACTIONS), row(NUM_ACTIONS)],
        out_shape=[jax.ShapeDtypeStruct((B,), f32),
                   jax.ShapeDtypeStruct((B, NUM_ACTIONS), f32),
                   jax.ShapeDtypeStruct((B, NUM_ACTIONS), f32)],
        compiler_params=pltpu.CompilerParams(
            dimension_semantics=("parallel",)),
    )(sp2, abit, tep, mvp, numeric, *wargs)
    return (v, p0, p1)
